# baseline (speedup 1.0000x reference)
import jax
import jax.numpy as jnp
from jax import lax
from jax.experimental import pallas as pl
from jax.experimental.pallas import tpu as pltpu
from jax.experimental.pallas import tpu_sc as plsc

B = 4096
N_FIELDS = 24
VOCAB = 100000
D = 64
HIST = 50

NC, NS = 2, 16
NW = NC * NS
TOK_ROWS = N_FIELDS * D
TPW = TOK_ROWS // NW
SPW = D // NW
OUT_D = TOK_ROWS + D + 1
NVREG = B // 16
H = 50048


def _body(tok_ids, seq_ids, feat, tok_tab, seq_tab, out,
          row_v, ids_v, ids2_v, out_v, sem_lo, sem_hi, sem_i, sem_o):
    wid = lax.axis_index("s") * NC + lax.axis_index("c")
    r0 = wid * TPW

    def tok_seg(f, lo, hi):
        pltpu.sync_copy(tok_ids.at[pl.ds(f * B, B)], ids_v)

        def tok_r(i, c):
            r = r0 + i
            d_lo = pltpu.async_copy(tok_tab.at[r, pl.ds(0, H)],
                                    row_v.at[pl.ds(0, H)], sem_lo)
            d_hi = pltpu.async_copy(tok_tab.at[r, pl.ds(0, VOCAB)],
                                    row_v, sem_hi)

            @pl.when(i > lo)
            def _():
                pltpu.make_async_copy(out_v, out.at[r - 1, pl.ds(0, B)],
                                      sem_o).wait()

            d_lo.wait()
            zero = jnp.zeros((16,), jnp.float32)

            def g_lo(v8, c2):
                for u in range(8):
                    sl = pl.ds(v8 * 128 + u * 16, 16)
                    ids = ids_v[sl]
                    g = plsc.load_gather(row_v, [jnp.minimum(ids, H - 1)])
                    out_v[sl] = jnp.where(ids < H, g, zero)
                return c2

            lax.fori_loop(0, NVREG // 8, g_lo, 0)
            d_hi.wait()

            def g_hi(v8, c2):
                for u in range(8):
                    sl = pl.ds(v8 * 128 + u * 16, 16)
                    ids = ids_v[sl]
                    g = plsc.load_gather(row_v, [jnp.maximum(ids, H)])
                    out_v[sl] = out_v[sl] + jnp.where(ids >= H, g, zero)
                return c2

            lax.fori_loop(0, NVREG // 8, g_hi, 0)
            pltpu.async_copy(out_v, out.at[r, pl.ds(0, B)], sem_o)
            return c

        lax.fori_loop(lo, hi, tok_r, 0)

        @pl.when(hi > lo)
        def _():
            pltpu.make_async_copy(out_v, out.at[r0 + hi - 1, pl.ds(0, B)],
                                  sem_o).wait()

    f0 = r0 // D
    n1 = jnp.minimum((f0 + 1) * D - r0, TPW)
    tok_seg(f0, 0, n1)
    tok_seg(jnp.minimum(f0 + 1, N_FIELDS - 1), n1, TPW)

    def seq_d(j, c):
        d = SPW * wid + j
        pltpu.sync_copy(seq_tab.at[d, pl.ds(0, VOCAB)], row_v)
        z = jnp.zeros((16,), jnp.float32)

        def zz(v8, c3):
            for u in range(8):
                out_v[pl.ds(v8 * 128 + u * 16, 16)] = z
            return c3

        lax.fori_loop(0, NVREG // 8, zz, 0)
        pltpu.async_copy(seq_ids.at[pl.ds(0, B)], ids_v, sem_i)

        def seq_tt(tt, c2):
            t0 = 2 * tt
            pltpu.make_async_copy(seq_ids.at[pl.ds(t0 * B, B)], ids_v,
                                  sem_i).wait()
            d1 = pltpu.async_copy(seq_ids.at[pl.ds((t0 + 1) * B, B)], ids2_v,
                                  sem_i)
            def ga(v8, c3):
                for u in range(8):
                    sl = pl.ds(v8 * 128 + u * 16, 16)
                    plsc.addupdate(out_v.at[sl],
                                   plsc.load_gather(row_v, [ids_v[sl]]))
                return c3

            lax.fori_loop(0, NVREG // 8, ga, 0)
            d1.wait()
            tn = jnp.minimum(t0 + 2, HIST - 1)
            pltpu.async_copy(seq_ids.at[pl.ds(tn * B, B)], ids_v, sem_i)
            def gb(v8, c3):
                for u in range(8):
                    sl = pl.ds(v8 * 128 + u * 16, 16)
                    plsc.addupdate(out_v.at[sl],
                                   plsc.load_gather(row_v, [ids2_v[sl]]))
                return c3

            lax.fori_loop(0, NVREG // 8, gb, 0)
            return c2

        lax.fori_loop(0, HIST // 2, seq_tt, 0)
        pltpu.make_async_copy(seq_ids.at[pl.ds(0, B)], ids_v, sem_i).wait()
        pltpu.sync_copy(out_v, out.at[TOK_ROWS + d, pl.ds(0, B)])
        return c

    lax.fori_loop(0, SPW, seq_d, 0)

    @pl.when(wid == 0)
    def _():
        pltpu.sync_copy(feat, out_v)
        pltpu.sync_copy(out_v, out.at[(OUT_D - 1) + wid // NW, pl.ds(0, B)])


@jax.jit
def _sc_embed(tok_ids_f, seq_ids_f, feat, tok_tab_t, seq_tab_t):
    mesh = plsc.VectorSubcoreMesh(core_axis_name="c", subcore_axis_name="s")
    fn = pl.kernel(
        _body,
        out_type=jax.ShapeDtypeStruct((OUT_D, B), jnp.float32),
        mesh=mesh,
        compiler_params=pltpu.CompilerParams(
            use_tc_tiling_on_sc=True, needs_layout_passes=False),
        scratch_types=[
            pltpu.VMEM((VOCAB,), jnp.float32),
            pltpu.VMEM((B,), jnp.int32),
            pltpu.VMEM((B,), jnp.int32),
            pltpu.VMEM((B,), jnp.float32),
            pltpu.SemaphoreType.DMA,
            pltpu.SemaphoreType.DMA,
            pltpu.SemaphoreType.DMA,
            pltpu.SemaphoreType.DMA,
        ],
    )
    return fn(tok_ids_f, seq_ids_f, feat, tok_tab_t, seq_tab_t)


def kernel(token_ids, seq_ids, float_feat, token_tables, seq_table):
    tok_tab_t = jnp.transpose(token_tables, (0, 2, 1)).reshape(TOK_ROWS, VOCAB)
    seq_tab_t = jnp.transpose(seq_table, (1, 0))
    tok_ids_f = jnp.transpose(token_ids.astype(jnp.int32)).reshape(B * N_FIELDS)
    seq_ids_f = jnp.transpose(seq_ids.astype(jnp.int32)).reshape(B * HIST)
    out_t = _sc_embed(tok_ids_f, seq_ids_f, float_feat.astype(jnp.float32),
                      tok_tab_t, seq_tab_t)
    return jnp.transpose(out_t)


# speedup vs baseline: 1.0746x; 1.0746x over previous
import jax
import jax.numpy as jnp
from jax import lax
from jax.experimental import pallas as pl
from jax.experimental.pallas import tpu as pltpu
from jax.experimental.pallas import tpu_sc as plsc

B = 4096
N_FIELDS = 24
VOCAB = 100000
D = 64
HIST = 50

NC, NS = 2, 16
NW = NC * NS
TOK_ROWS = N_FIELDS * D
TPW = TOK_ROWS // NW
SPW = D // NW
OUT_D = TOK_ROWS + D + 1
NVREG = B // 16
H = 50048


def _body(tok_ids, seq_ids, feat, tok_tab, seq_tab, out,
          row_v, ids_v, ids2_v, out_v, sem_lo, sem_hi, sem_i, sem_o):
    wid = lax.axis_index("s") * NC + lax.axis_index("c")
    r0 = wid * TPW

    def tok_seg(f, lo, hi):
        pltpu.sync_copy(tok_ids.at[pl.ds(f * B, B)], ids_v)

        def tok_r(i, c):
            r = r0 + i
            pltpu.sync_copy(tok_tab.at[r, pl.ds(0, VOCAB)], row_v)

            @pl.when(i > lo)
            def _():
                pltpu.make_async_copy(out_v, out.at[r - 1, pl.ds(0, B)],
                                      sem_o).wait()

            for v in range(NVREG):
                sl = pl.ds(v * 16, 16)
                out_v[sl] = plsc.load_gather(row_v, [ids_v[sl]])
            pltpu.async_copy(out_v, out.at[r, pl.ds(0, B)], sem_o)
            return c

        lax.fori_loop(lo, hi, tok_r, 0)

        @pl.when(hi > lo)
        def _():
            pltpu.make_async_copy(out_v, out.at[r0 + hi - 1, pl.ds(0, B)],
                                  sem_o).wait()

    f0 = r0 // D
    n1 = jnp.minimum((f0 + 1) * D - r0, TPW)
    tok_seg(f0, 0, n1)
    tok_seg(jnp.minimum(f0 + 1, N_FIELDS - 1), n1, TPW)

    def seq_d(j, c):
        d = SPW * wid + j
        pltpu.sync_copy(seq_tab.at[d, pl.ds(0, VOCAB)], row_v)
        z = jnp.zeros((16,), jnp.float32)
        for v in range(NVREG):
            out_v[pl.ds(v * 16, 16)] = z
        pltpu.async_copy(seq_ids.at[pl.ds(0, B)], ids_v, sem_i)

        def seq_tt(tt, c2):
            t0 = 2 * tt
            pltpu.make_async_copy(seq_ids.at[pl.ds(t0 * B, B)], ids_v,
                                  sem_i).wait()
            d1 = pltpu.async_copy(seq_ids.at[pl.ds((t0 + 1) * B, B)], ids2_v,
                                  sem_i)
            for v in range(NVREG):
                sl = pl.ds(v * 16, 16)
                plsc.addupdate(out_v.at[sl],
                               plsc.load_gather(row_v, [ids_v[sl]]))
            d1.wait()
            tn = jnp.minimum(t0 + 2, HIST - 1)
            pltpu.async_copy(seq_ids.at[pl.ds(tn * B, B)], ids_v, sem_i)
            for v in range(NVREG):
                sl = pl.ds(v * 16, 16)
                plsc.addupdate(out_v.at[sl],
                               plsc.load_gather(row_v, [ids2_v[sl]]))
            return c2

        lax.fori_loop(0, HIST // 2, seq_tt, 0)
        pltpu.make_async_copy(seq_ids.at[pl.ds(0, B)], ids_v, sem_i).wait()
        pltpu.sync_copy(out_v, out.at[TOK_ROWS + d, pl.ds(0, B)])
        return c

    lax.fori_loop(0, SPW, seq_d, 0)

    @pl.when(wid == 0)
    def _():
        pltpu.sync_copy(feat, out_v)
        pltpu.sync_copy(out_v, out.at[(OUT_D - 1) + wid // NW, pl.ds(0, B)])


@jax.jit
def _sc_embed(tok_ids_f, seq_ids_f, feat, tok_tab_t, seq_tab_t):
    mesh = plsc.VectorSubcoreMesh(core_axis_name="c", subcore_axis_name="s")
    fn = pl.kernel(
        _body,
        out_type=jax.ShapeDtypeStruct((OUT_D, B), jnp.float32),
        mesh=mesh,
        compiler_params=pltpu.CompilerParams(
            use_tc_tiling_on_sc=True, needs_layout_passes=False),
        scratch_types=[
            pltpu.VMEM((VOCAB,), jnp.float32),
            pltpu.VMEM((B,), jnp.int32),
            pltpu.VMEM((B,), jnp.int32),
            pltpu.VMEM((B,), jnp.float32),
            pltpu.SemaphoreType.DMA,
            pltpu.SemaphoreType.DMA,
            pltpu.SemaphoreType.DMA,
            pltpu.SemaphoreType.DMA,
        ],
    )
    return fn(tok_ids_f, seq_ids_f, feat, tok_tab_t, seq_tab_t)


def kernel(token_ids, seq_ids, float_feat, token_tables, seq_table):
    tok_tab_t = jnp.transpose(token_tables, (0, 2, 1)).reshape(TOK_ROWS, VOCAB)
    seq_tab_t = jnp.transpose(seq_table, (1, 0))
    tok_ids_f = jnp.transpose(token_ids.astype(jnp.int32)).reshape(B * N_FIELDS)
    seq_ids_f = jnp.transpose(seq_ids.astype(jnp.int32)).reshape(B * HIST)
    out_t = _sc_embed(tok_ids_f, seq_ids_f, float_feat.astype(jnp.float32),
                      tok_tab_t, seq_tab_t)
    return jnp.transpose(out_t)
